# baseline (speedup 1.0000x reference)
import functools

import jax
import jax.numpy as jnp
from jax import lax
from jax.experimental import pallas as pl
from jax.experimental.pallas import tpu as pltpu
from jax.experimental.pallas import tpu_sc as plsc

NUM_FREQ = 4
IN_DIM = 4
OUT_DIM = 4
NUM_BINS = 34
ROW = OUT_DIM * IN_DIM * NUM_FREQ
OI = ROW // NUM_FREQ
SROW = ROW + 1
E = 800000
ETILES = E // 128

NC = 2
NS = 16
NW = NC * NS
L = 16

TPW = ETILES // NW
XTRA = ETILES - TPW * NW
ST = 5
NSUP = TPW // ST
DMAX = (TPW + 1) * 128

_MAGIC = 8388608.0


def _bins_from_dists(d):
    x = jnp.clip((d - 2.4) / 0.4, 0.0, 33.0)
    r = (x + _MAGIC) - _MAGIC
    return r.astype(jnp.int32)


_mesh = plsc.VectorSubcoreMesh(core_axis_name="c", subcore_axis_name="s")


@functools.partial(
    pl.kernel,
    mesh=_mesh,
    out_type=jax.ShapeDtypeStruct((OI, ETILES, NUM_FREQ, 128), jnp.float32),
    scratch_types=[
        pltpu.VMEM((NUM_BINS * SROW,), jnp.float32),
        pltpu.VMEM((DMAX,), jnp.float32),
        pltpu.VMEM((ST * 128,), jnp.int32),
        [pltpu.VMEM((OI, ST, NUM_FREQ, 128), jnp.float32)
         for _ in range(2)],
        [pltpu.SemaphoreType.DMA for _ in range(2)],
    ],
    compiler_params=pltpu.CompilerParams(use_tc_tiling_on_sc=False,
                                         needs_layout_passes=False),
)
def _radial_sc(dists_hbm, table_hbm, out_hbm, tbl_v, d_v, bin_v, tbuf, sem_w):
    wid = lax.axis_index("s") * NC + lax.axis_index("c")
    has_extra = wid < XTRA
    t0 = wid * TPW + jnp.minimum(wid, XTRA)

    pltpu.sync_copy(table_hbm, tbl_v)
    pltpu.sync_copy(dists_hbm.at[pl.ds(t0 * 128, TPW * 128)],
                    d_v.at[pl.ds(0, TPW * 128)])

    @pl.when(has_extra)
    def _():
        pltpu.sync_copy(dists_hbm.at[pl.ds((t0 + TPW) * 128, 128)],
                        d_v.at[pl.ds(TPW * 128, 128)])

    def gather_tile(b, t, loff):
        for eg in range(128 // L):
            base = bin_v[pl.ds(loff + eg * L, L)] * SROW
            idxs = [base + f for f in range(NUM_FREQ)]
            for oi in range(OI):
                for f in range(NUM_FREQ):
                    v = plsc.load_gather(tbl_v, [idxs[f]])
                    tbuf[b][oi, t, f, pl.ds(eg * L, L)] = v
                if oi < OI - 1:
                    idxs = [i + NUM_FREQ for i in idxs]

    def process(g, b, nt):
        @pl.loop(0, nt * 128 // L)
        def _(i):
            bin_v[pl.ds(i * L, L)] = _bins_from_dists(
                d_v[pl.ds(g * (ST * 128) + i * L, L)])

        @pl.loop(0, nt)
        def _(t):
            gather_tile(b, t, t * 128)

        for oi in range(OI):
            pltpu.async_copy(tbuf[b].at[oi, pl.ds(0, nt)],
                             out_hbm.at[oi, pl.ds(t0 + g * ST, nt)], sem_w[b])

    def drain(b, nt):
        pltpu.make_async_copy(tbuf[b].at[:, pl.ds(0, nt)],
                              out_hbm.at[:, pl.ds(0, nt)], sem_w[b]).wait()

    @pl.loop(0, NSUP - 1, step=2)
    def _(j):
        for b in range(2):
            k = j + b

            @pl.when(k >= 2)
            def _():
                drain(b, ST)

            process(k, b, ST)

    drain(0, ST)
    process(NSUP - 1, 0, ST)
    drain(1, ST)

    @pl.when(has_extra)
    def _():
        process(NSUP, 1, 1)

    drain(0, ST)

    @pl.when(has_extra)
    def _():
        drain(1, 1)


def kernel(dists, bin_embedding):
    tpad = jnp.pad(bin_embedding, ((0, 0), (0, 1))).reshape(NUM_BINS * SROW)
    x = _radial_sc(dists.reshape(E), tpad)
    x = x.reshape(OUT_DIM, IN_DIM, ETILES, NUM_FREQ, 128)
    x = x.transpose(2, 4, 0, 1, 3).reshape(E, OUT_DIM, IN_DIM, NUM_FREQ)
    return x[:, :, None, :, None, :]


# speedup vs baseline: 1.3909x; 1.3909x over previous
import functools

import jax
import jax.numpy as jnp
from jax import lax
from jax.experimental import pallas as pl
from jax.experimental.pallas import tpu as pltpu
from jax.experimental.pallas import tpu_sc as plsc

NUM_FREQ = 4
IN_DIM = 4
OUT_DIM = 4
NUM_BINS = 34
ROW = OUT_DIM * IN_DIM * NUM_FREQ
E = 800000
ETILES = E // 128

NC = 2
NS = 16
NW = NC * NS
L = 16

ST = 5
EPB = ST * 128
NSUP = ETILES // ST
NIT = -(-NSUP // NW)

_MAGIC = 8388608.0


def _bins_from_dists(d):
    x = jnp.clip((d - 2.4) / 0.4, 0.0, 33.0)
    r = (x + _MAGIC) - _MAGIC
    return r.astype(jnp.int32)


_mesh = plsc.VectorSubcoreMesh(core_axis_name="c", subcore_axis_name="s")


@functools.partial(
    pl.kernel,
    mesh=_mesh,
    out_type=jax.ShapeDtypeStruct((ROW // NUM_FREQ, ETILES, NUM_FREQ, 128),
                                  jnp.float32),
    scratch_types=[
        pltpu.VMEM((NUM_BINS * (ROW + 1),), jnp.float32),
        [pltpu.VMEM((EPB,), jnp.float32) for _ in range(2)],
        [pltpu.VMEM((EPB,), jnp.int32) for _ in range(2)],
        [pltpu.VMEM((ROW // NUM_FREQ, ST, NUM_FREQ, 128), jnp.float32)
         for _ in range(2)],
        [pltpu.SemaphoreType.DMA for _ in range(2)],
        [pltpu.SemaphoreType.DMA for _ in range(2)],
    ],
    compiler_params=pltpu.CompilerParams(use_tc_tiling_on_sc=False,
                                         needs_layout_passes=False),
)
def _radial_sc(dists_hbm, table_hbm, out_hbm, tbl_v, d_v, bin_v, tbuf, sem_w,
               sem_d):
    wid = lax.axis_index("s") * NC + lax.axis_index("c")

    pltpu.sync_copy(table_hbm, tbl_v)

    def prefetch(s, b):
        pltpu.async_copy(dists_hbm.at[pl.ds(s * EPB, EPB)], d_v[b], sem_d[b])

    prefetch(wid, 0)
    prefetch(wid + NW, 1)

    def drain(b):
        pltpu.make_async_copy(
            tbuf[b], out_hbm.at[:, pl.ds(0, ST)], sem_w[b]).wait()

    def process(s, b):
        pltpu.make_async_copy(dists_hbm.at[pl.ds(0, EPB)], d_v[b],
                              sem_d[b]).wait()

        @pl.loop(0, EPB // L)
        def _(g):
            bin_v[b][pl.ds(g * L, L)] = _bins_from_dists(d_v[b][pl.ds(g * L, L)])

        @pl.when(s + 2 * NW < NSUP)
        def _():
            prefetch(s + 2 * NW, b)

        @pl.loop(0, ST)
        def _(t):
            @pl.loop(0, 128 // L)
            def _(eg):
                idx = bin_v[b][pl.ds(t * 128 + eg * L, L)] * (ROW + 1)
                for c in range(ROW):
                    v = plsc.load_gather(tbl_v, [idx])
                    tbuf[b][c // NUM_FREQ, t, c % NUM_FREQ,
                            pl.ds(eg * L, L)] = v
                    if c < ROW - 1:
                        idx = idx + 1

        pltpu.async_copy(tbuf[b], out_hbm.at[:, pl.ds(s * ST, ST)], sem_w[b])

    @pl.loop(0, NIT, step=2)
    def _(j):
        for b in range(2):
            k = j + b
            s = wid + k * NW

            @pl.when(k >= 2)
            def _():
                drain(b)

            @pl.when(s < NSUP)
            def _():
                process(s, b)

    drain((NIT - 2) % 2)

    @pl.when(wid < NSUP - (NIT - 1) * NW)
    def _():
        drain((NIT - 1) % 2)


def kernel(dists, bin_embedding):
    tpad = jnp.pad(bin_embedding, ((0, 0), (0, 1))).reshape(NUM_BINS * (ROW + 1))
    x = _radial_sc(dists.reshape(E), tpad)
    x = x.reshape(OUT_DIM, IN_DIM, ETILES, NUM_FREQ, 128)
    x = x.transpose(2, 4, 0, 1, 3).reshape(E, OUT_DIM, IN_DIM, NUM_FREQ)
    return x[:, :, None, :, None, :]


# speedup vs baseline: 1.3914x; 1.0003x over previous
import functools

import jax
import jax.numpy as jnp
from jax import lax
from jax.experimental import pallas as pl
from jax.experimental.pallas import tpu as pltpu
from jax.experimental.pallas import tpu_sc as plsc

NUM_FREQ = 4
IN_DIM = 4
OUT_DIM = 4
NUM_BINS = 34
ROW = OUT_DIM * IN_DIM * NUM_FREQ
E = 800000
ETILES = E // 128

NC = 2
NS = 16
NW = NC * NS
L = 16

ST = 5
EPB = ST * 128
NSUP = ETILES // ST
NIT = -(-NSUP // NW)

_MAGIC = 8388608.0


def _bins_from_dists(d):
    x = jnp.clip((d - 2.4) / 0.4, 0.0, 33.0)
    r = (x + _MAGIC) - _MAGIC
    return r.astype(jnp.int32)


_mesh = plsc.VectorSubcoreMesh(core_axis_name="c", subcore_axis_name="s")


@functools.partial(
    pl.kernel,
    mesh=_mesh,
    out_type=jax.ShapeDtypeStruct((ROW // NUM_FREQ, ETILES, NUM_FREQ, 128),
                                  jnp.float32),
    scratch_types=[
        pltpu.VMEM((NUM_BINS * (ROW + 1),), jnp.float32),
        [pltpu.VMEM((EPB,), jnp.float32) for _ in range(2)],
        [pltpu.VMEM((EPB,), jnp.int32) for _ in range(2)],
        [pltpu.VMEM((ROW // NUM_FREQ, ST, NUM_FREQ, 128), jnp.float32)
         for _ in range(2)],
        [pltpu.SemaphoreType.DMA for _ in range(2)],
        [pltpu.SemaphoreType.DMA for _ in range(2)],
    ],
    compiler_params=pltpu.CompilerParams(use_tc_tiling_on_sc=False,
                                         needs_layout_passes=False),
)
def _radial_sc(dists_hbm, table_hbm, out_hbm, tbl_v, d_v, bin_v, tbuf, sem_w,
               sem_d):
    wid = lax.axis_index("s") * NC + lax.axis_index("c")

    pltpu.sync_copy(table_hbm, tbl_v)

    def prefetch(s, b):
        pltpu.async_copy(dists_hbm.at[pl.ds(s * EPB, EPB)], d_v[b], sem_d[b])

    prefetch(wid, 0)
    prefetch(wid + NW, 1)

    def drain(b):
        pltpu.make_async_copy(
            tbuf[b], out_hbm.at[:, pl.ds(0, ST)], sem_w[b]).wait()

    def process(s, b):
        pltpu.make_async_copy(dists_hbm.at[pl.ds(0, EPB)], d_v[b],
                              sem_d[b]).wait()

        @pl.loop(0, EPB // L)
        def _(g):
            bin_v[b][pl.ds(g * L, L)] = _bins_from_dists(d_v[b][pl.ds(g * L, L)])

        @pl.when(s + 2 * NW < NSUP)
        def _():
            prefetch(s + 2 * NW, b)

        @pl.loop(0, ST)
        def _(t):
            @pl.loop(0, 128 // L)
            def _(eg):
                base = bin_v[b][pl.ds(t * 128 + eg * L, L)] * (ROW + 1)
                idxs = [base + f for f in range(NUM_FREQ)]
                for oi in range(ROW // NUM_FREQ):
                    for f in range(NUM_FREQ):
                        v = plsc.load_gather(tbl_v, [idxs[f]])
                        tbuf[b][oi, t, f, pl.ds(eg * L, L)] = v
                    if oi < ROW // NUM_FREQ - 1:
                        idxs = [i + NUM_FREQ for i in idxs]

        pltpu.async_copy(tbuf[b], out_hbm.at[:, pl.ds(s * ST, ST)], sem_w[b])

    @pl.loop(0, NIT, step=2)
    def _(j):
        for b in range(2):
            k = j + b
            s = wid + k * NW

            @pl.when(k >= 2)
            def _():
                drain(b)

            @pl.when(s < NSUP)
            def _():
                process(s, b)

    drain((NIT - 2) % 2)

    @pl.when(wid < NSUP - (NIT - 1) * NW)
    def _():
        drain((NIT - 1) % 2)


def kernel(dists, bin_embedding):
    tpad = jnp.pad(bin_embedding, ((0, 0), (0, 1))).reshape(NUM_BINS * (ROW + 1))
    x = _radial_sc(dists.reshape(E), tpad)
    x = x.reshape(OUT_DIM, IN_DIM, ETILES, NUM_FREQ, 128)
    x = x.transpose(2, 4, 0, 1, 3).reshape(E, OUT_DIM, IN_DIM, NUM_FREQ)
    return x[:, :, None, :, None, :]
